# baseline (speedup 1.0000x reference)
import jax
import jax.numpy as jnp
from jax import lax
from jax.experimental import pallas as pl
from jax.experimental.pallas import tpu as pltpu

BATCH = 16384
NF = 100
NFIELD = 26
EMB = 16
B_CH = 4096
GRID = BATCH // B_CH


def _body(fi_ref, vt_ref, x_ref, out_ref, et_ref):
    @pl.when(pl.program_id(0) == 0)
    def _build_et():
        fi_row = jnp.broadcast_to(fi_ref[...], (EMB, NF))
        et = jnp.zeros((EMB, NF), jnp.float32)
        for c in range(NFIELD):
            et = jnp.where(fi_row == float(c), vt_ref[:, c:c + 1], et)
        et_ref[...] = et

    for f in range(NF):
        x_row = x_ref[f:f + 1, :]
        e_col = et_ref[:, f:f + 1]
        out_ref[f] = e_col * x_row


def kernel(inputs, V, field_index):
    x_t = inputs.T
    v_t = V.T
    fi_f = field_index.astype(jnp.float32).reshape(1, NF)
    out_t = pl.pallas_call(
        _body,
        grid=(GRID,),
        in_specs=[
            pl.BlockSpec((1, NF), lambda i: (0, 0)),
            pl.BlockSpec((EMB, NFIELD), lambda i: (0, 0)),
            pl.BlockSpec((NF, B_CH), lambda i: (0, i)),
        ],
        out_specs=pl.BlockSpec((NF, EMB, B_CH), lambda i: (0, 0, i)),
        out_shape=jax.ShapeDtypeStruct((NF, EMB, BATCH), jnp.float32),
        scratch_shapes=[pltpu.VMEM((EMB, NF), jnp.float32)],
        compiler_params=pltpu.CompilerParams(
            dimension_semantics=("arbitrary",),
        ),
    )(fi_f, v_t, x_t)
    return jnp.transpose(out_t, (2, 0, 1))


# speedup vs baseline: 1.0869x; 1.0869x over previous
import jax
import jax.numpy as jnp
from jax import lax
from jax.experimental import pallas as pl
from jax.experimental.pallas import tpu as pltpu

BATCH = 16384
NF = 100
NFIELD = 26
EMB = 16
B_CH = 2048
GRID = BATCH // B_CH


def _body(fi_ref, vt_ref, x_ref, out_ref, et_ref):
    @pl.when(pl.program_id(0) == 0)
    def _build_et():
        fi_row = jnp.broadcast_to(fi_ref[...], (EMB, NF))
        et = jnp.zeros((EMB, NF), jnp.float32)
        for c in range(NFIELD):
            et = jnp.where(fi_row == float(c), vt_ref[:, c:c + 1], et)
        et_ref[...] = et

    for f in range(NF):
        x_row = x_ref[f:f + 1, :]
        e_col = et_ref[:, f:f + 1]
        out_ref[f] = e_col * x_row


def kernel(inputs, V, field_index):
    x_t = inputs.T
    v_t = V.T
    fi_f = field_index.astype(jnp.float32).reshape(1, NF)
    out_t = pl.pallas_call(
        _body,
        grid=(GRID,),
        in_specs=[
            pl.BlockSpec((1, NF), lambda i: (0, 0)),
            pl.BlockSpec((EMB, NFIELD), lambda i: (0, 0)),
            pl.BlockSpec((NF, B_CH), lambda i: (0, i)),
        ],
        out_specs=pl.BlockSpec((NF, EMB, B_CH), lambda i: (0, 0, i)),
        out_shape=jax.ShapeDtypeStruct((NF, EMB, BATCH), jnp.float32),
        scratch_shapes=[pltpu.VMEM((EMB, NF), jnp.float32)],
        compiler_params=pltpu.CompilerParams(
            dimension_semantics=("arbitrary",),
        ),
    )(fi_f, v_t, x_t)
    return jnp.transpose(out_t, (2, 0, 1))
